# Initial kernel scaffold; baseline (speedup 1.0000x reference)
import functools

import jax
import jax.numpy as jnp
from jax import lax
from jax.experimental import pallas as pl
from jax.experimental.pallas import tpu as pltpu
from jax.experimental.pallas import tpu_sc as plsc

N = 10000
E = 160000
H = 256
NCLS = 40
G = 128

NPAD = 10016
JROW = 10008
STRIPE = NPAD // 16
ECHUNK = 128
NCHUNK = 80
DCHUNK = 40
EPAD = 16 * NCHUNK * ECHUNK
RB = 1000
GRID = N // RB

_mesh = plsc.VectorSubcoreMesh(core_axis_name="c", subcore_axis_name="s")


def _deg_body(dst_h, z_h, out_h, idx_v, ones_v, acc):
    c = lax.axis_index("c")
    s = lax.axis_index("s")
    wid = c * 16 + s
    pltpu.sync_copy(z_h.at[pl.ds(s * STRIPE, STRIPE)],
                    acc.at[pl.ds(s * STRIPE, STRIPE)])
    pltpu.sync_copy(dst_h.at[wid], idx_v)

    def fill(i, carry):
        ones_v[i, :] = jnp.ones((16,), jnp.float32)
        return carry
    lax.fori_loop(0, ECHUNK, fill, 0)
    plsc.subcore_barrier()

    def step(j, carry):
        pltpu.sync_copy(ones_v, acc.at[idx_v.at[j]], add=True)
        return carry
    lax.fori_loop(0, DCHUNK, step, 0)
    plsc.subcore_barrier()
    pltpu.sync_copy(acc.at[pl.ds(s * STRIPE, STRIPE)],
                    out_h.at[c, pl.ds(s * STRIPE, STRIPE)])


_deg = pl.kernel(
    _deg_body,
    out_type=jax.ShapeDtypeStruct((2, NPAD, 16), jnp.float32),
    mesh=_mesh,
    scratch_types=[
        pltpu.VMEM((DCHUNK, ECHUNK), jnp.int32),
        pltpu.VMEM((ECHUNK, 16), jnp.float32),
        pltpu.VMEM_SHARED((NPAD, 16), jnp.float32),
    ],
)


def _scat_body(hs0, hs1, src_h, dst_h, z_h, out_h,
               sidx, didx, buf, acc, gsem, ssem):
    c = lax.axis_index("c")
    s = lax.axis_index("s")
    pltpu.sync_copy(z_h.at[pl.ds(s * STRIPE, STRIPE)],
                    acc.at[pl.ds(s * STRIPE, STRIPE)])
    pltpu.sync_copy(src_h.at[s], sidx)
    pltpu.sync_copy(dst_h.at[s], didx)
    plsc.subcore_barrier()

    def run(table):
        for b in range(2):
            pltpu.async_copy(table.at[sidx.at[b]], buf.at[b], gsem.at[b])

        def outer(o, carry):
            for b in range(2):
                j = o * 2 + b
                pltpu.make_async_copy(table.at[sidx.at[j]], buf.at[b],
                                      gsem.at[b]).wait()
                pltpu.async_copy(buf.at[b], acc.at[didx.at[j]], ssem.at[b],
                                 add=True)

                @pl.when(o < NCHUNK // 2 - 1)
                def _():
                    pltpu.make_async_copy(buf.at[b], acc.at[didx.at[j]],
                                          ssem.at[b]).wait()
                    pltpu.async_copy(table.at[sidx.at[j + 2]], buf.at[b],
                                     gsem.at[b])
            return carry
        lax.fori_loop(0, NCHUNK // 2, outer, 0)
        for b in range(2):
            j = NCHUNK - 2 + b
            pltpu.make_async_copy(buf.at[b], acc.at[didx.at[j]],
                                  ssem.at[b]).wait()

    @pl.when(c == 0)
    def _():
        run(hs0)

    @pl.when(c == 1)
    def _():
        run(hs1)

    plsc.subcore_barrier()
    pltpu.sync_copy(acc.at[pl.ds(s * STRIPE, STRIPE)],
                    out_h.at[c, pl.ds(s * STRIPE, STRIPE)])


_scatter = pl.kernel(
    _scat_body,
    out_type=jax.ShapeDtypeStruct((2, NPAD, 128), jnp.float32),
    mesh=_mesh,
    scratch_types=[
        pltpu.VMEM((NCHUNK, ECHUNK), jnp.int32),
        pltpu.VMEM((NCHUNK, ECHUNK), jnp.int32),
        pltpu.VMEM((2, ECHUNK, 128), jnp.float32),
        pltpu.SemaphoreType.DMA((2,)),
        pltpu.SemaphoreType.DMA((2,)),
    ],
)


def _dinv_body(dp_ref, o_ref):
    d = dp_ref[0] + dp_ref[1]
    d1 = d[:, 0:1] + 1.0
    o_ref[...] = jnp.broadcast_to(lax.rsqrt(d1), (RB, 128))


_dinv = pl.pallas_call(
    _dinv_body,
    grid=(GRID,),
    in_specs=[pl.BlockSpec((2, RB, 16), lambda i: (0, i, 0))],
    out_specs=pl.BlockSpec((RB, 128), lambda i: (i, 0)),
    out_shape=jax.ShapeDtypeStruct((NPAD, 128), jnp.float32),
)


def _l1_body(x_ref, w_ref, dv_ref, h0_ref, h1_ref):
    h = jnp.dot(x_ref[...], w_ref[...], preferred_element_type=jnp.float32)
    dv = dv_ref[...]
    h0_ref[...] = h[:, :128] * dv
    h1_ref[...] = h[:, 128:] * dv


_l1 = pl.pallas_call(
    _l1_body,
    grid=(GRID,),
    in_specs=[
        pl.BlockSpec((RB, H), lambda i: (i, 0)),
        pl.BlockSpec((H, H), lambda i: (0, 0)),
        pl.BlockSpec((RB, 128), lambda i: (i, 0)),
    ],
    out_specs=[
        pl.BlockSpec((RB, 128), lambda i: (i, 0)),
        pl.BlockSpec((RB, 128), lambda i: (i, 0)),
    ],
    out_shape=[
        jax.ShapeDtypeStruct((NPAD, 128), jnp.float32),
        jax.ShapeDtypeStruct((NPAD, 128), jnp.float32),
    ],
)


def _ep_body(a_ref, p0_ref, p1_ref, dv_ref, b_ref, w_ref, h0_ref, h1_ref):
    dv = dv_ref[...]
    u0 = jnp.maximum(dv * (a_ref[0] + p0_ref[...]) + b_ref[0:1, :128], 0.0)
    u1 = jnp.maximum(dv * (a_ref[1] + p1_ref[...]) + b_ref[0:1, 128:], 0.0)
    h = (jnp.dot(u0, w_ref[:128, :], preferred_element_type=jnp.float32)
         + jnp.dot(u1, w_ref[128:, :], preferred_element_type=jnp.float32))
    h0_ref[...] = h[:, :128] * dv
    h1_ref[...] = h[:, 128:] * dv


_ep = pl.pallas_call(
    _ep_body,
    grid=(GRID,),
    in_specs=[
        pl.BlockSpec((2, RB, 128), lambda i: (0, i, 0)),
        pl.BlockSpec((RB, 128), lambda i: (i, 0)),
        pl.BlockSpec((RB, 128), lambda i: (i, 0)),
        pl.BlockSpec((RB, 128), lambda i: (i, 0)),
        pl.BlockSpec((8, H), lambda i: (0, 0)),
        pl.BlockSpec((H, H), lambda i: (0, 0)),
    ],
    out_specs=[
        pl.BlockSpec((RB, 128), lambda i: (i, 0)),
        pl.BlockSpec((RB, 128), lambda i: (i, 0)),
    ],
    out_shape=[
        jax.ShapeDtypeStruct((NPAD, 128), jnp.float32),
        jax.ShapeDtypeStruct((NPAD, 128), jnp.float32),
    ],
)


def _fin_body(a_ref, p0_ref, p1_ref, dv_ref, b_ref, bt_ref, wl_ref, bl_ref,
              o_ref, pool):
    i = pl.program_id(0)
    dv = dv_ref[...]
    u0 = jnp.maximum(dv * (a_ref[0] + p0_ref[...]) + b_ref[0:1, :128], 0.0)
    u1 = jnp.maximum(dv * (a_ref[1] + p1_ref[...]) + b_ref[0:1, 128:], 0.0)
    bt = bt_ref[0]
    gi = lax.broadcasted_iota(jnp.int32, (G, RB), 0)
    oh = (gi == bt).astype(jnp.float32)

    @pl.when(i == 0)
    def _():
        pool[...] = jnp.zeros((G, H), jnp.float32)

    pool[:, :128] += jnp.dot(oh, u0, preferred_element_type=jnp.float32)
    pool[:, 128:] += jnp.dot(oh, u1, preferred_element_type=jnp.float32)

    @pl.when(i == GRID - 1)
    def _():
        p = jnp.maximum(pool[...], 0.0)
        logits = (jnp.dot(p, wl_ref[...], preferred_element_type=jnp.float32)
                  + bl_ref[0:1, :])
        mask = lax.broadcasted_iota(jnp.int32, (G, 128), 1) < NCLS
        lg = jnp.where(mask, logits, -1e30)
        m = jnp.max(lg, axis=1, keepdims=True)
        lse = jnp.log(jnp.sum(jnp.exp(lg - m), axis=1, keepdims=True)) + m
        o_ref[...] = lg - lse


_fin = pl.pallas_call(
    _fin_body,
    grid=(GRID,),
    in_specs=[
        pl.BlockSpec((2, RB, 128), lambda i: (0, i, 0)),
        pl.BlockSpec((RB, 128), lambda i: (i, 0)),
        pl.BlockSpec((RB, 128), lambda i: (i, 0)),
        pl.BlockSpec((RB, 128), lambda i: (i, 0)),
        pl.BlockSpec((8, H), lambda i: (0, 0)),
        pl.BlockSpec((1, 1, RB), lambda i: (i, 0, 0)),
        pl.BlockSpec((H, 128), lambda i: (0, 0)),
        pl.BlockSpec((8, 128), lambda i: (0, 0)),
    ],
    out_specs=pl.BlockSpec((G, 128), lambda i: (0, 0)),
    out_shape=jax.ShapeDtypeStruct((G, 128), jnp.float32),
    scratch_shapes=[pltpu.VMEM((G, H), jnp.float32)],
)


def kernel(x, edge_index, batch, W1, b1, W2, b2, W3, b3, Wl, bl):
    src = edge_index[0].astype(jnp.int32)
    dst = edge_index[1].astype(jnp.int32)
    pad = jnp.full((EPAD - E,), JROW, jnp.int32)
    srcp = jnp.concatenate([src, pad])
    dstp = jnp.concatenate([dst, pad])
    src_sc = srcp.reshape(16, NCHUNK, ECHUNK)
    dst_sc = dstp.reshape(16, NCHUNK, ECHUNK)
    dst_deg = dstp.reshape(32, DCHUNK, ECHUNK)
    z16 = jnp.zeros((NPAD, 16), jnp.float32)
    z128 = jnp.zeros((NPAD, 128), jnp.float32)
    b1b = jnp.broadcast_to(b1, (8, H))
    b2b = jnp.broadcast_to(b2, (8, H))
    b3b = jnp.broadcast_to(b3, (8, H))
    wlp = jnp.zeros((H, 128), jnp.float32).at[:, :NCLS].set(Wl)
    blp = jnp.broadcast_to(jnp.pad(bl, (0, 128 - NCLS)), (8, 128))
    bt3 = batch.astype(jnp.int32).reshape(GRID, 1, RB)

    degp = _deg(dst_deg, z16)
    dinv = _dinv(degp)
    hs0, hs1 = _l1(x, W1, dinv)
    agg = _scatter(hs0, hs1, src_sc, dst_sc, z128)
    hs0, hs1 = _ep(agg, hs0, hs1, dinv, b1b, W2)
    agg = _scatter(hs0, hs1, src_sc, dst_sc, z128)
    hs0, hs1 = _ep(agg, hs0, hs1, dinv, b2b, W3)
    agg = _scatter(hs0, hs1, src_sc, dst_sc, z128)
    out = _fin(agg, hs0, hs1, dinv, b3b, bt3, wlp, blp)
    return out[:, :NCLS]


# speedup vs baseline: 6.2146x; 6.2146x over previous
import functools

import jax
import jax.numpy as jnp
from jax import lax
from jax.experimental import pallas as pl
from jax.experimental.pallas import tpu as pltpu
from jax.experimental.pallas import tpu_sc as plsc

N = 10000
E = 160000
H = 256
NCLS = 40
G = 128

NPAD = 10240
JROW = 10008
STRIPE = NPAD // 16
ECHUNK = 128
NCHUNK = 80
DCHUNK = 40
EPAD = 16 * NCHUNK * ECHUNK
RB = 1000
GRID = N // RB


def _deg_body(dst_h, z_h, out_h, idx_v, ones_v, acc):
    c = lax.axis_index("c")
    s = lax.axis_index("s")
    wid = c * 16 + s
    pltpu.sync_copy(z_h.at[pl.ds(s * STRIPE, STRIPE)],
                    acc.at[pl.ds(s * STRIPE, STRIPE)])
    pltpu.sync_copy(dst_h.at[wid], idx_v)

    def fill(i, carry):
        ones_v[i, :] = jnp.ones((16,), jnp.float32)
        return carry
    lax.fori_loop(0, ECHUNK, fill, 0)
    plsc.subcore_barrier()

    def step(j, carry):
        pltpu.sync_copy(ones_v, acc.at[idx_v.at[j]], add=True)
        return carry
    lax.fori_loop(0, DCHUNK, step, 0)
    plsc.subcore_barrier()
    pltpu.sync_copy(acc.at[pl.ds(s * STRIPE, STRIPE)],
                    out_h.at[c, pl.ds(s * STRIPE, STRIPE)])


@functools.cache
def _get_deg():
    return pl.kernel(
        _deg_body,
        out_type=jax.ShapeDtypeStruct((2, NPAD, 16), jnp.float32),
        mesh=plsc.VectorSubcoreMesh(core_axis_name="c", subcore_axis_name="s"),
        scratch_types=[
            pltpu.VMEM((DCHUNK, ECHUNK), jnp.int32),
            pltpu.VMEM((ECHUNK, 16), jnp.float32),
            pltpu.VMEM_SHARED((NPAD, 16), jnp.float32),
        ],
    )


def _scat_body(hs0, hs1, src_h, dst_h, z_h, out_h,
               sidx, didx, buf, acc):
    c = lax.axis_index("c")
    s = lax.axis_index("s")
    pltpu.sync_copy(z_h.at[pl.ds(s * STRIPE, STRIPE)],
                    acc.at[pl.ds(s * STRIPE, STRIPE)])
    pltpu.sync_copy(src_h.at[s], sidx)
    pltpu.sync_copy(dst_h.at[s], didx)
    plsc.subcore_barrier()

    def run(table):
        def step(j, carry):
            pltpu.sync_copy(table.at[sidx.at[j]], buf)
            pltpu.sync_copy(buf, acc.at[didx.at[j]], add=True)
            return carry
        lax.fori_loop(0, NCHUNK, step, 0)

    @pl.when(c == 0)
    def _():
        run(hs0)

    @pl.when(c == 1)
    def _():
        run(hs1)

    plsc.subcore_barrier()
    pltpu.sync_copy(acc.at[pl.ds(s * STRIPE, STRIPE)],
                    out_h.at[c, pl.ds(s * STRIPE, STRIPE)])


@functools.cache
def _get_scatter():
    return pl.kernel(
        _scat_body,
        out_type=jax.ShapeDtypeStruct((2, NPAD, 128), jnp.float32),
        mesh=plsc.VectorSubcoreMesh(core_axis_name="c", subcore_axis_name="s"),
        scratch_types=[
            pltpu.VMEM((NCHUNK, ECHUNK), jnp.int32),
            pltpu.VMEM((NCHUNK, ECHUNK), jnp.int32),
            pltpu.VMEM((ECHUNK, 128), jnp.float32),
            pltpu.VMEM_SHARED((NPAD, 128), jnp.float32),
        ],
    )


def _dinv_body(dp_ref, o_ref):
    d = dp_ref[0] + dp_ref[1]
    d1 = d[:, 0:1] + 1.0
    o_ref[...] = jnp.broadcast_to(lax.rsqrt(d1), (RB, 128))


_dinv = pl.pallas_call(
    _dinv_body,
    grid=(GRID,),
    in_specs=[pl.BlockSpec((2, RB, 16), lambda i: (0, i, 0))],
    out_specs=pl.BlockSpec((RB, 128), lambda i: (i, 0)),
    out_shape=jax.ShapeDtypeStruct((NPAD, 128), jnp.float32),
)


def _l1_body(x_ref, w_ref, dv_ref, h0_ref, h1_ref):
    h = jnp.dot(x_ref[...], w_ref[...], preferred_element_type=jnp.float32)
    dv = dv_ref[...]
    h0_ref[...] = h[:, :128] * dv
    h1_ref[...] = h[:, 128:] * dv


_l1 = pl.pallas_call(
    _l1_body,
    grid=(GRID,),
    in_specs=[
        pl.BlockSpec((RB, H), lambda i: (i, 0)),
        pl.BlockSpec((H, H), lambda i: (0, 0)),
        pl.BlockSpec((RB, 128), lambda i: (i, 0)),
    ],
    out_specs=[
        pl.BlockSpec((RB, 128), lambda i: (i, 0)),
        pl.BlockSpec((RB, 128), lambda i: (i, 0)),
    ],
    out_shape=[
        jax.ShapeDtypeStruct((NPAD, 128), jnp.float32),
        jax.ShapeDtypeStruct((NPAD, 128), jnp.float32),
    ],
)


def _ep_body(a_ref, p0_ref, p1_ref, dv_ref, b_ref, w_ref, h0_ref, h1_ref):
    dv = dv_ref[...]
    u0 = jnp.maximum(dv * (a_ref[0] + p0_ref[...]) + b_ref[0:1, :128], 0.0)
    u1 = jnp.maximum(dv * (a_ref[1] + p1_ref[...]) + b_ref[0:1, 128:], 0.0)
    h = (jnp.dot(u0, w_ref[:128, :], preferred_element_type=jnp.float32)
         + jnp.dot(u1, w_ref[128:, :], preferred_element_type=jnp.float32))
    h0_ref[...] = h[:, :128] * dv
    h1_ref[...] = h[:, 128:] * dv


_ep = pl.pallas_call(
    _ep_body,
    grid=(GRID,),
    in_specs=[
        pl.BlockSpec((2, RB, 128), lambda i: (0, i, 0)),
        pl.BlockSpec((RB, 128), lambda i: (i, 0)),
        pl.BlockSpec((RB, 128), lambda i: (i, 0)),
        pl.BlockSpec((RB, 128), lambda i: (i, 0)),
        pl.BlockSpec((8, H), lambda i: (0, 0)),
        pl.BlockSpec((H, H), lambda i: (0, 0)),
    ],
    out_specs=[
        pl.BlockSpec((RB, 128), lambda i: (i, 0)),
        pl.BlockSpec((RB, 128), lambda i: (i, 0)),
    ],
    out_shape=[
        jax.ShapeDtypeStruct((NPAD, 128), jnp.float32),
        jax.ShapeDtypeStruct((NPAD, 128), jnp.float32),
    ],
)


def _fin_body(a_ref, p0_ref, p1_ref, dv_ref, b_ref, bt_ref, wl_ref, bl_ref,
              o_ref, pool):
    i = pl.program_id(0)
    dv = dv_ref[...]
    u0 = jnp.maximum(dv * (a_ref[0] + p0_ref[...]) + b_ref[0:1, :128], 0.0)
    u1 = jnp.maximum(dv * (a_ref[1] + p1_ref[...]) + b_ref[0:1, 128:], 0.0)
    bt = bt_ref[0]
    gi = lax.broadcasted_iota(jnp.int32, (G, RB), 0)
    oh = (gi == bt).astype(jnp.float32)

    @pl.when(i == 0)
    def _():
        pool[...] = jnp.zeros((G, H), jnp.float32)

    pool[:, :128] += jnp.dot(oh, u0, preferred_element_type=jnp.float32)
    pool[:, 128:] += jnp.dot(oh, u1, preferred_element_type=jnp.float32)

    @pl.when(i == GRID - 1)
    def _():
        p = jnp.maximum(pool[...], 0.0)
        logits = (jnp.dot(p, wl_ref[...], preferred_element_type=jnp.float32)
                  + bl_ref[0:1, :])
        mask = lax.broadcasted_iota(jnp.int32, (G, 128), 1) < NCLS
        lg = jnp.where(mask, logits, -1e30)
        m = jnp.max(lg, axis=1, keepdims=True)
        lse = jnp.log(jnp.sum(jnp.exp(lg - m), axis=1, keepdims=True)) + m
        o_ref[...] = lg - lse


_fin = pl.pallas_call(
    _fin_body,
    grid=(GRID,),
    in_specs=[
        pl.BlockSpec((2, RB, 128), lambda i: (0, i, 0)),
        pl.BlockSpec((RB, 128), lambda i: (i, 0)),
        pl.BlockSpec((RB, 128), lambda i: (i, 0)),
        pl.BlockSpec((RB, 128), lambda i: (i, 0)),
        pl.BlockSpec((8, H), lambda i: (0, 0)),
        pl.BlockSpec((1, 1, RB), lambda i: (i, 0, 0)),
        pl.BlockSpec((H, 128), lambda i: (0, 0)),
        pl.BlockSpec((8, 128), lambda i: (0, 0)),
    ],
    out_specs=pl.BlockSpec((G, 128), lambda i: (0, 0)),
    out_shape=jax.ShapeDtypeStruct((G, 128), jnp.float32),
    scratch_shapes=[pltpu.VMEM((G, H), jnp.float32)],
)


def kernel(x, edge_index, batch, W1, b1, W2, b2, W3, b3, Wl, bl):
    src = edge_index[0].astype(jnp.int32)
    dst = edge_index[1].astype(jnp.int32)
    pad = jnp.full((EPAD - E,), JROW, jnp.int32)
    srcp = jnp.concatenate([src, pad])
    dstp = jnp.concatenate([dst, pad])
    src_sc = srcp.reshape(16, NCHUNK, ECHUNK)
    dst_sc = dstp.reshape(16, NCHUNK, ECHUNK)
    dst_deg = dstp.reshape(32, DCHUNK, ECHUNK)
    z16 = jnp.zeros((NPAD, 16), jnp.float32)
    z128 = jnp.zeros((NPAD, 128), jnp.float32)
    b1b = jnp.broadcast_to(b1, (8, H))
    b2b = jnp.broadcast_to(b2, (8, H))
    b3b = jnp.broadcast_to(b3, (8, H))
    wlp = jnp.zeros((H, 128), jnp.float32).at[:, :NCLS].set(Wl)
    blp = jnp.broadcast_to(jnp.pad(bl, (0, 128 - NCLS)), (8, 128))
    bt3 = batch.astype(jnp.int32).reshape(GRID, 1, RB)

    _scatter = _get_scatter()
    degp = _get_deg()(dst_deg, z16)
    dinv = _dinv(degp)
    hs0, hs1 = _l1(x, W1, dinv)
    agg = _scatter(hs0, hs1, src_sc, dst_sc, z128)
    hs0, hs1 = _ep(agg, hs0, hs1, dinv, b1b, W2)
    agg = _scatter(hs0, hs1, src_sc, dst_sc, z128)
    hs0, hs1 = _ep(agg, hs0, hs1, dinv, b2b, W3)
    agg = _scatter(hs0, hs1, src_sc, dst_sc, z128)
    out = _fin(agg, hs0, hs1, dinv, b3b, bt3, wlp, blp)
    return out[:, :NCLS]


# speedup vs baseline: 6.7262x; 1.0823x over previous
import functools

import jax
import jax.numpy as jnp
from jax import lax
from jax.experimental import pallas as pl
from jax.experimental.pallas import tpu as pltpu
from jax.experimental.pallas import tpu_sc as plsc

N = 10000
E = 160000
H = 256
NCLS = 40
G = 128

NPAD = 10240
JROW = 10008
STRIPE = NPAD // 16
ECHUNK = 128
NCHUNK = 80
DCHUNK = 40
EPAD = 16 * NCHUNK * ECHUNK
RB = 1000
GRID = N // RB


def _deg_body(dst_h, z_h, out_h, idx_v, ones_v, acc):
    c = lax.axis_index("c")
    s = lax.axis_index("s")
    wid = c * 16 + s
    pltpu.sync_copy(z_h.at[pl.ds(s * STRIPE, STRIPE)],
                    acc.at[pl.ds(s * STRIPE, STRIPE)])
    pltpu.sync_copy(dst_h.at[wid], idx_v)

    def fill(i, carry):
        ones_v[i, :] = jnp.ones((16,), jnp.float32)
        return carry
    lax.fori_loop(0, ECHUNK, fill, 0)
    plsc.subcore_barrier()

    def step(j, carry):
        pltpu.sync_copy(ones_v, acc.at[idx_v.at[j]], add=True)
        return carry
    lax.fori_loop(0, DCHUNK, step, 0)
    plsc.subcore_barrier()
    pltpu.sync_copy(acc.at[pl.ds(s * STRIPE, STRIPE)],
                    out_h.at[c, pl.ds(s * STRIPE, STRIPE)])


@functools.cache
def _get_deg():
    return pl.kernel(
        _deg_body,
        out_type=jax.ShapeDtypeStruct((2, NPAD, 16), jnp.float32),
        mesh=plsc.VectorSubcoreMesh(core_axis_name="c", subcore_axis_name="s"),
        scratch_types=[
            pltpu.VMEM((DCHUNK, ECHUNK), jnp.int32),
            pltpu.VMEM((ECHUNK, 16), jnp.float32),
            pltpu.VMEM_SHARED((NPAD, 16), jnp.float32),
        ],
    )


BLK = 8
NBLK = NCHUNK // BLK


def _scat_body(hs0, hs1, sd_h, z_h, out_h, idxblk, buf, acc,
               isem, gsem, ssem):
    c = lax.axis_index("c")
    s = lax.axis_index("s")
    pltpu.sync_copy(z_h.at[pl.ds(s * STRIPE, STRIPE)],
                    acc.at[pl.ds(s * STRIPE, STRIPE)])
    plsc.subcore_barrier()

    def run(table):
        pltpu.async_copy(sd_h.at[s, pl.ds(0, BLK)], idxblk.at[0], isem.at[0])
        pltpu.make_async_copy(sd_h.at[s, pl.ds(0, BLK)], idxblk.at[0],
                              isem.at[0]).wait()
        pltpu.async_copy(sd_h.at[s, pl.ds(BLK, BLK)], idxblk.at[1], isem.at[1])
        for b in range(2):
            pltpu.async_copy(table.at[idxblk.at[0, b, 0]], buf.at[b],
                             gsem.at[b])

        def blk_body(k, carry):
            kb = lax.rem(k, 2)
            kb1 = lax.rem(k + 1, 2)
            for p in range(BLK):
                b = p % 2
                pltpu.make_async_copy(table.at[idxblk.at[kb, p, 0]],
                                      buf.at[b], gsem.at[b]).wait()
                pltpu.async_copy(buf.at[b], acc.at[idxblk.at[kb, p, 1]],
                                 ssem.at[b], add=True)
                if p > 0:
                    pltpu.make_async_copy(buf.at[1 - b],
                                          acc.at[idxblk.at[kb, p - 1, 1]],
                                          ssem.at[1 - b]).wait()
                    if p < BLK - 1:
                        pltpu.async_copy(table.at[idxblk.at[kb, p + 1, 0]],
                                         buf.at[1 - b], gsem.at[1 - b])
                else:
                    @pl.when(k > 0)
                    def _():
                        pltpu.make_async_copy(
                            buf.at[1], acc.at[idxblk.at[kb1, BLK - 1, 1]],
                            ssem.at[1]).wait()
                        pltpu.async_copy(table.at[idxblk.at[kb, 1, 0]],
                                         buf.at[1], gsem.at[1])

                        @pl.when(k < NBLK - 1)
                        def _():
                            pltpu.async_copy(
                                sd_h.at[s, pl.ds((k + 1) * BLK, BLK)],
                                idxblk.at[kb1], isem.at[kb1])
                if p == BLK - 1:
                    @pl.when(k < NBLK - 1)
                    def _():
                        pltpu.make_async_copy(
                            sd_h.at[s, pl.ds((k + 1) * BLK, BLK)],
                            idxblk.at[kb1], isem.at[kb1]).wait()
                        pltpu.async_copy(table.at[idxblk.at[kb1, 0, 0]],
                                         buf.at[1 - b], gsem.at[1 - b])
            return carry
        lax.fori_loop(0, NBLK, blk_body, 0)
        kb_last = lax.rem(NBLK - 1, 2)
        pltpu.make_async_copy(buf.at[1], acc.at[idxblk.at[kb_last, BLK - 1, 1]],
                              ssem.at[1]).wait()

    @pl.when(c == 0)
    def _():
        run(hs0)

    @pl.when(c == 1)
    def _():
        run(hs1)

    plsc.subcore_barrier()
    pltpu.sync_copy(acc.at[pl.ds(s * STRIPE, STRIPE)],
                    out_h.at[c, pl.ds(s * STRIPE, STRIPE)])


@functools.cache
def _get_scatter():
    return pl.kernel(
        _scat_body,
        out_type=jax.ShapeDtypeStruct((2, NPAD, 128), jnp.float32),
        mesh=plsc.VectorSubcoreMesh(core_axis_name="c", subcore_axis_name="s"),
        scratch_types=[
            pltpu.VMEM((2, BLK, 2, ECHUNK), jnp.int32),
            pltpu.VMEM((2, ECHUNK, 128), jnp.float32),
            pltpu.VMEM_SHARED((NPAD, 128), jnp.float32),
            pltpu.SemaphoreType.DMA((2,)),
            pltpu.SemaphoreType.DMA((2,)),
            pltpu.SemaphoreType.DMA((2,)),
        ],
    )


def _dinv_body(dp_ref, o_ref):
    d = dp_ref[0] + dp_ref[1]
    d1 = d[:, 0:1] + 1.0
    o_ref[...] = jnp.broadcast_to(lax.rsqrt(d1), (RB, 128))


_dinv = pl.pallas_call(
    _dinv_body,
    grid=(GRID,),
    in_specs=[pl.BlockSpec((2, RB, 16), lambda i: (0, i, 0))],
    out_specs=pl.BlockSpec((RB, 128), lambda i: (i, 0)),
    out_shape=jax.ShapeDtypeStruct((NPAD, 128), jnp.float32),
)


def _l1_body(x_ref, w_ref, dv_ref, h0_ref, h1_ref):
    h = jnp.dot(x_ref[...], w_ref[...], preferred_element_type=jnp.float32)
    dv = dv_ref[...]
    h0_ref[...] = h[:, :128] * dv
    h1_ref[...] = h[:, 128:] * dv


_l1 = pl.pallas_call(
    _l1_body,
    grid=(GRID,),
    in_specs=[
        pl.BlockSpec((RB, H), lambda i: (i, 0)),
        pl.BlockSpec((H, H), lambda i: (0, 0)),
        pl.BlockSpec((RB, 128), lambda i: (i, 0)),
    ],
    out_specs=[
        pl.BlockSpec((RB, 128), lambda i: (i, 0)),
        pl.BlockSpec((RB, 128), lambda i: (i, 0)),
    ],
    out_shape=[
        jax.ShapeDtypeStruct((NPAD, 128), jnp.float32),
        jax.ShapeDtypeStruct((NPAD, 128), jnp.float32),
    ],
)


def _ep_body(a_ref, p0_ref, p1_ref, dv_ref, b_ref, w_ref, h0_ref, h1_ref):
    dv = dv_ref[...]
    u0 = jnp.maximum(dv * (a_ref[0] + p0_ref[...]) + b_ref[0:1, :128], 0.0)
    u1 = jnp.maximum(dv * (a_ref[1] + p1_ref[...]) + b_ref[0:1, 128:], 0.0)
    h = (jnp.dot(u0, w_ref[:128, :], preferred_element_type=jnp.float32)
         + jnp.dot(u1, w_ref[128:, :], preferred_element_type=jnp.float32))
    h0_ref[...] = h[:, :128] * dv
    h1_ref[...] = h[:, 128:] * dv


_ep = pl.pallas_call(
    _ep_body,
    grid=(GRID,),
    in_specs=[
        pl.BlockSpec((2, RB, 128), lambda i: (0, i, 0)),
        pl.BlockSpec((RB, 128), lambda i: (i, 0)),
        pl.BlockSpec((RB, 128), lambda i: (i, 0)),
        pl.BlockSpec((RB, 128), lambda i: (i, 0)),
        pl.BlockSpec((8, H), lambda i: (0, 0)),
        pl.BlockSpec((H, H), lambda i: (0, 0)),
    ],
    out_specs=[
        pl.BlockSpec((RB, 128), lambda i: (i, 0)),
        pl.BlockSpec((RB, 128), lambda i: (i, 0)),
    ],
    out_shape=[
        jax.ShapeDtypeStruct((NPAD, 128), jnp.float32),
        jax.ShapeDtypeStruct((NPAD, 128), jnp.float32),
    ],
)


def _fin_body(a_ref, p0_ref, p1_ref, dv_ref, b_ref, bt_ref, wl_ref, bl_ref,
              o_ref, pool):
    i = pl.program_id(0)
    dv = dv_ref[...]
    u0 = jnp.maximum(dv * (a_ref[0] + p0_ref[...]) + b_ref[0:1, :128], 0.0)
    u1 = jnp.maximum(dv * (a_ref[1] + p1_ref[...]) + b_ref[0:1, 128:], 0.0)
    bt = bt_ref[0]
    gi = lax.broadcasted_iota(jnp.int32, (G, RB), 0)
    oh = (gi == bt).astype(jnp.float32)

    @pl.when(i == 0)
    def _():
        pool[...] = jnp.zeros((G, H), jnp.float32)

    pool[:, :128] += jnp.dot(oh, u0, preferred_element_type=jnp.float32)
    pool[:, 128:] += jnp.dot(oh, u1, preferred_element_type=jnp.float32)

    @pl.when(i == GRID - 1)
    def _():
        p = jnp.maximum(pool[...], 0.0)
        logits = (jnp.dot(p, wl_ref[...], preferred_element_type=jnp.float32)
                  + bl_ref[0:1, :])
        mask = lax.broadcasted_iota(jnp.int32, (G, 128), 1) < NCLS
        lg = jnp.where(mask, logits, -1e30)
        m = jnp.max(lg, axis=1, keepdims=True)
        lse = jnp.log(jnp.sum(jnp.exp(lg - m), axis=1, keepdims=True)) + m
        o_ref[...] = lg - lse


_fin = pl.pallas_call(
    _fin_body,
    grid=(GRID,),
    in_specs=[
        pl.BlockSpec((2, RB, 128), lambda i: (0, i, 0)),
        pl.BlockSpec((RB, 128), lambda i: (i, 0)),
        pl.BlockSpec((RB, 128), lambda i: (i, 0)),
        pl.BlockSpec((RB, 128), lambda i: (i, 0)),
        pl.BlockSpec((8, H), lambda i: (0, 0)),
        pl.BlockSpec((1, 1, RB), lambda i: (i, 0, 0)),
        pl.BlockSpec((H, 128), lambda i: (0, 0)),
        pl.BlockSpec((8, 128), lambda i: (0, 0)),
    ],
    out_specs=pl.BlockSpec((G, 128), lambda i: (0, 0)),
    out_shape=jax.ShapeDtypeStruct((G, 128), jnp.float32),
    scratch_shapes=[pltpu.VMEM((G, H), jnp.float32)],
)


def kernel(x, edge_index, batch, W1, b1, W2, b2, W3, b3, Wl, bl):
    src = edge_index[0].astype(jnp.int32)
    dst = edge_index[1].astype(jnp.int32)
    pad = jnp.full((EPAD - E,), JROW, jnp.int32)
    srcp = jnp.concatenate([src, pad])
    dstp = jnp.concatenate([dst, pad])
    sd = jnp.stack([srcp.reshape(16, NCHUNK, ECHUNK),
                    dstp.reshape(16, NCHUNK, ECHUNK)], axis=2)
    dst_deg = dstp.reshape(32, DCHUNK, ECHUNK)
    z16 = jnp.zeros((NPAD, 16), jnp.float32)
    z128 = jnp.zeros((NPAD, 128), jnp.float32)
    b1b = jnp.broadcast_to(b1, (8, H))
    b2b = jnp.broadcast_to(b2, (8, H))
    b3b = jnp.broadcast_to(b3, (8, H))
    wlp = jnp.zeros((H, 128), jnp.float32).at[:, :NCLS].set(Wl)
    blp = jnp.broadcast_to(jnp.pad(bl, (0, 128 - NCLS)), (8, 128))
    bt3 = batch.astype(jnp.int32).reshape(GRID, 1, RB)

    _scatter = _get_scatter()
    degp = _get_deg()(dst_deg, z16)
    dinv = _dinv(degp)
    hs0, hs1 = _l1(x, W1, dinv)
    agg = _scatter(hs0, hs1, sd, z128)
    hs0, hs1 = _ep(agg, hs0, hs1, dinv, b1b, W2)
    agg = _scatter(hs0, hs1, sd, z128)
    hs0, hs1 = _ep(agg, hs0, hs1, dinv, b2b, W3)
    agg = _scatter(hs0, hs1, sd, z128)
    out = _fin(agg, hs0, hs1, dinv, b3b, bt3, wlp, blp)
    return out[:, :NCLS]
